# baseline (speedup 1.0000x reference)
import functools

import jax
import jax.numpy as jnp
from jax import lax
from jax.experimental import pallas as pl
from jax.experimental.pallas import tpu as pltpu
from jax.experimental.pallas import tpu_sc as plsc

_NC = 2
_NS = 16
_NW = _NC * _NS


def _prep_body(Wm2, Wm3, bm2, bm3, Wa2, Wa3, Wu2, Wu3, bu2, bu3,
               Wm23, bm23, wa23b, Wu23, bu23):
    f32 = jnp.float32
    Wm23[...] = jnp.dot(Wm2[...], Wm3[...], preferred_element_type=f32)
    bm23[...] = jnp.dot(bm2[...], Wm3[...], preferred_element_type=f32) + bm3[...]
    wa23 = jnp.dot(Wa2[...], Wa3[...], preferred_element_type=f32)
    wa23b[...] = jnp.dot(wa23, jnp.ones((1, wa23.shape[0]), f32),
                         preferred_element_type=f32)
    Wu23[...] = jnp.dot(Wu2[...], Wu3[...], preferred_element_type=f32)
    bu23[...] = jnp.dot(bu2[...], Wu3[...], preferred_element_type=f32) + bu3[...]


def _prep(Wm2, Wm3, bm2, bm3, Wa2, Wa3, Wu2, Wu3, bu2, bu3, interpret=False):
    H, M = Wm3.shape
    A = Wa2.shape[0]
    U, O = Wu3.shape
    return pl.pallas_call(
        _prep_body,
        out_shape=[
            jax.ShapeDtypeStruct((H, M), jnp.float32),
            jax.ShapeDtypeStruct((1, M), jnp.float32),
            jax.ShapeDtypeStruct((A, A), jnp.float32),
            jax.ShapeDtypeStruct((U, O), jnp.float32),
            jax.ShapeDtypeStruct((1, O), jnp.float32),
        ],
        interpret=interpret,
    )(Wm2, Wm3, bm2.reshape(1, H), bm3.reshape(1, M), Wa2, Wa3,
      Wu2, Wu3, bu2.reshape(1, U), bu3.reshape(1, O))


def _make_gather(N, D, E, C, interpret=False):
    EW = E // _NW
    NCH = EW // C
    DP = D
    dt = jnp.float32
    mesh = plsc.VectorSubcoreMesh(core_axis_name="c", subcore_axis_name="s",
                                  num_cores=_NC, num_subcores=_NS)

    @functools.partial(
        pl.kernel,
        out_type=(jax.ShapeDtypeStruct((E, DP), dt),
                  jax.ShapeDtypeStruct((E, DP), dt)),
        mesh=mesh,
        scratch_types=[
            pltpu.VMEM((NCH, C), jnp.int32),
            pltpu.VMEM((NCH, C), jnp.int32),
            pltpu.VMEM((C, DP), dt),
            pltpu.VMEM((C, DP), dt),
            pltpu.VMEM((C, DP), dt),
            pltpu.VMEM((C, DP), dt),
        ] + [pltpu.SemaphoreType.DMA] * 8,
        interpret=interpret,
    )
    def gather(x_hbm, s_hbm, r_hbm, xs_hbm, xr_hbm,
               sbuf, rbuf, a0, b0, a1, b1,
               sga0, sgb0, sga1, sgb1, swa0, swb0, swa1, swb1):
        cid = lax.axis_index("c")
        sid = lax.axis_index("s")
        wid = sid * _NC + cid
        pltpu.sync_copy(s_hbm.at[wid], sbuf)
        pltpu.sync_copy(r_hbm.at[wid], rbuf)
        base = wid * EW

        def body(p, carry):
            j0 = 2 * p
            j1 = j0 + 1

            @pl.when(p > 0)
            def _():
                pltpu.make_async_copy(a0, xs_hbm.at[pl.ds(base, C)], swa0).wait()
                pltpu.make_async_copy(b0, xr_hbm.at[pl.ds(base, C)], swb0).wait()

            ga = pltpu.async_copy(x_hbm.at[sbuf.at[j0]], a0, sga0)
            gb = pltpu.async_copy(x_hbm.at[rbuf.at[j0]], b0, sgb0)

            @pl.when(p > 0)
            def _():
                pltpu.make_async_copy(a1, xs_hbm.at[pl.ds(base, C)], swa1).wait()
                pltpu.make_async_copy(b1, xr_hbm.at[pl.ds(base, C)], swb1).wait()

            ga.wait()
            gb.wait()
            pltpu.async_copy(a0, xs_hbm.at[pl.ds(base + j0 * C, C)], swa0)
            pltpu.async_copy(b0, xr_hbm.at[pl.ds(base + j0 * C, C)], swb0)
            ga1 = pltpu.async_copy(x_hbm.at[sbuf.at[j1]], a1, sga1)
            gb1 = pltpu.async_copy(x_hbm.at[rbuf.at[j1]], b1, sgb1)
            ga1.wait()
            gb1.wait()
            pltpu.async_copy(a1, xs_hbm.at[pl.ds(base + j1 * C, C)], swa1)
            pltpu.async_copy(b1, xr_hbm.at[pl.ds(base + j1 * C, C)], swb1)
            return carry

        lax.fori_loop(0, NCH // 2, body, 0)

        pltpu.make_async_copy(a0, xs_hbm.at[pl.ds(base, C)], swa0).wait()
        pltpu.make_async_copy(b0, xr_hbm.at[pl.ds(base, C)], swb0).wait()
        pltpu.make_async_copy(a1, xs_hbm.at[pl.ds(base, C)], swa1).wait()
        pltpu.make_async_copy(b1, xr_hbm.at[pl.ds(base, C)], swb1).wait()

        if NCH % 2:
            jt = NCH - 1
            ga = pltpu.async_copy(x_hbm.at[sbuf.at[jt]], a0, sga0)
            gb = pltpu.async_copy(x_hbm.at[rbuf.at[jt]], b0, sgb0)
            ga.wait()
            gb.wait()
            pltpu.sync_copy(a0, xs_hbm.at[pl.ds(base + jt * C, C)])
            pltpu.sync_copy(b0, xr_hbm.at[pl.ds(base + jt * C, C)])

    return gather


def _msg_body(ea, xs, xr, We, Ws, Wr, bm1, Wm23, bm23, Wa1, ba1, wa23b,
              em_ref, e_ref):
    f32 = jnp.float32
    dflt = jax.lax.Precision.DEFAULT
    dot = lambda a, b: jnp.dot(a, b, preferred_element_type=f32,
                               precision=dflt)
    h = dot(ea[...], We[...])
    h = h + dot(xs[...], Ws[...])
    h = h + dot(xr[...], Wr[...])
    h = jnp.maximum(h + bm1[...], 0.0)
    msgs = dot(h, Wm23[...]) + bm23[...]
    g1 = jnp.maximum(dot(msgs, Wa1[...]) + ba1[...], 0.0)
    gate = dot(g1, wa23b[...])
    ev = jnp.exp(gate)
    em_ref[...] = msgs * ev
    e_ref[...] = ev[:, 0]


def _messages(ea, xs, xr, We, Ws, Wr, bm1, Wm23, bm23, Wa1, ba1, wa23,
              interpret=False):
    E, DE = ea.shape
    DP = xs.shape[1]
    D = Wa1.shape[0]
    H = Ws.shape[1]
    TE = 512
    grid = E // TE
    full2 = lambda a, b: pl.BlockSpec((a, b), lambda i: (0, 0))
    return pl.pallas_call(
        _msg_body,
        grid=(grid,),
        in_specs=[
            pl.BlockSpec((TE, DE), lambda i: (i, 0)),
            pl.BlockSpec((TE, DP), lambda i: (i, 0)),
            pl.BlockSpec((TE, DP), lambda i: (i, 0)),
            full2(DE, H), full2(D, H), full2(D, H), full2(1, H),
            full2(H, D), full2(1, D), full2(D, D), full2(1, D), full2(D, D),
        ],
        out_specs=[
            pl.BlockSpec((TE, D), lambda i: (i, 0)),
            pl.BlockSpec((TE,), lambda i: (i,)),
        ],
        out_shape=[
            jax.ShapeDtypeStruct((E, D), jnp.float32),
            jax.ShapeDtypeStruct((E,), jnp.float32),
        ],
        compiler_params=pltpu.CompilerParams(
            dimension_semantics=("arbitrary",)),
        interpret=interpret,
    )(ea, xs, xr, We, Ws, Wr, bm1, Wm23, bm23, Wa1, ba1, wa23)


def _make_aggregate(N, D, E, NP, C, P, nph, init, interpret=False):
    EP = E // P
    EWP = EP // _NW
    NB = NP // _NS
    ZR = 64
    B2 = EWP // C
    mesh = plsc.VectorSubcoreMesh(core_axis_name="c", subcore_axis_name="s",
                                  num_cores=_NC, num_subcores=_NS)

    @functools.partial(
        pl.kernel,
        out_type=(jax.ShapeDtypeStruct((_NC, NP, D), jnp.float32),
                  jax.ShapeDtypeStruct((_NC, NP), jnp.float32)),
        mesh=mesh,
        scratch_types=[
            pltpu.VMEM((B2, C), jnp.int32),
            pltpu.VMEM((B2, C), jnp.float32),
            pltpu.VMEM((C, D), jnp.float32),
            pltpu.VMEM((C, D), jnp.float32),
            pltpu.VMEM((ZR, D), jnp.float32),
            pltpu.VMEM((NB,), jnp.float32),
            pltpu.VMEM_SHARED((NP, D), jnp.float32),
            pltpu.VMEM_SHARED((NP,), jnp.float32),
            pltpu.SemaphoreType.DMA,
            pltpu.SemaphoreType.DMA,
            pltpu.SemaphoreType.DMA,
        ],
        interpret=interpret,
    )
    def agg(*refs):
        ems = refs[:nph]
        e_hbm = refs[nph]
        r_hbm = refs[nph + 1]
        k0 = nph + 2
        if init:
            initA = refs[k0]
            initD = refs[k0 + 1]
            k0 += 2
        (aggr_out, den_out, rbuf, ebuf, r0, r1, zrows, dstage,
         aggr_sh, den_sh, sl0, sl1, se) = refs[k0:]
        cid = lax.axis_index("c")
        sid = lax.axis_index("s")
        wid = sid * _NC + cid
        zero16 = jnp.zeros((16,), jnp.float32)

        tb = sid * NB
        if init:
            pltpu.sync_copy(initD.at[cid, pl.ds(tb, NB)], dstage)
            pltpu.sync_copy(dstage, den_sh.at[pl.ds(tb, NB)])
            for k in range(NB // ZR):
                pltpu.sync_copy(initA.at[cid, pl.ds(tb + k * ZR, ZR)], zrows)
                pltpu.sync_copy(zrows, aggr_sh.at[pl.ds(tb + k * ZR, ZR)])
        else:
            def z1(i, c):
                dstage[pl.ds(i * 16, 16)] = zero16
                return c
            lax.fori_loop(0, NB // 16, z1, 0)

            def z2(t, c):
                zrows[t // (D // 16), pl.ds((t % (D // 16)) * 16, 16)] = zero16
                return c
            lax.fori_loop(0, ZR * (D // 16), z2, 0)

            for k in range(NB // ZR):
                pltpu.sync_copy(zrows, aggr_sh.at[pl.ds(tb + k * ZR, ZR)])
            pltpu.sync_copy(dstage, den_sh.at[pl.ds(tb, NB)])
        plsc.subcore_barrier()

        base = wid * EWP

        for b, em_hbm in enumerate(ems):
            pltpu.sync_copy(r_hbm.at[b, wid], rbuf)
            pltpu.sync_copy(e_hbm.at[b, wid], ebuf)

            def pair(p, c2, em_hbm=em_hbm):
                j0 = 2 * p
                j1 = j0 + 1
                ld0 = pltpu.async_copy(
                    em_hbm.at[pl.ds(base + j0 * C, C)], r0, sl0)
                ld1 = pltpu.async_copy(
                    em_hbm.at[pl.ds(base + j1 * C, C)], r1, sl1)
                pltpu.async_copy(ebuf.at[j0], den_sh.at[rbuf.at[j0]], se,
                                 add=True)
                pltpu.async_copy(ebuf.at[j1], den_sh.at[rbuf.at[j1]], se,
                                 add=True)
                ld0.wait()
                pltpu.sync_copy(r0, aggr_sh.at[rbuf.at[j0]], add=True)
                ld1.wait()
                pltpu.sync_copy(r1, aggr_sh.at[rbuf.at[j1]], add=True)
                return c2

            lax.fori_loop(0, B2 // 2, pair, 0)
            if B2 % 2:
                jt = B2 - 1
                pltpu.async_copy(ebuf.at[jt], den_sh.at[rbuf.at[jt]], se,
                                 add=True)
                pltpu.sync_copy(em_hbm.at[pl.ds(base + jt * C, C)], r0)
                pltpu.sync_copy(r0, aggr_sh.at[rbuf.at[jt]], add=True)

            def drain(k, c2):
                pltpu.make_async_copy(
                    ebuf.at[0], den_sh.at[rbuf.at[0]], se).wait()
                return c2

            lax.fori_loop(0, B2, drain, 0)
        plsc.subcore_barrier()

        pltpu.sync_copy(den_sh.at[pl.ds(tb, NB)], dstage)
        pltpu.sync_copy(dstage, den_out.at[cid, pl.ds(tb, NB)])
        for k in range(NB // ZR):
            pltpu.sync_copy(aggr_sh.at[pl.ds(tb + k * ZR, ZR)], zrows)
            pltpu.sync_copy(zrows, aggr_out.at[cid, pl.ds(tb + k * ZR, ZR)])

    return agg


def _upd_body(x, a0, a1, d0, d1, W1x, W1a, b1, W23, b23, o_ref):
    f32 = jnp.float32
    d = d0[...] + d1[...]
    pos = d > 0.0
    inv = jnp.where(pos, 1.0 / jnp.where(pos, d, 1.0), 0.0)
    aggr = (a0[...] + a1[...]) * inv
    u = jnp.dot(x[...], W1x[...], preferred_element_type=f32)
    u = u + jnp.dot(aggr, W1a[...], preferred_element_type=f32)
    u = jnp.maximum(u + b1[...], 0.0)
    o_ref[...] = jnp.dot(u, W23[...], preferred_element_type=f32) + b23[...]


def _update(x, a0, a1, d0, d1, W1x, W1a, b1, W23, b23, interpret=False):
    N, D = x.shape
    H = W1x.shape[1]
    O = W23.shape[1]
    TN = 1000
    grid = N // TN
    full2 = lambda a, b: pl.BlockSpec((a, b), lambda i: (0, 0))
    return pl.pallas_call(
        _upd_body,
        grid=(grid,),
        in_specs=[
            pl.BlockSpec((TN, D), lambda i: (i, 0)),
            pl.BlockSpec((TN, D), lambda i: (i, 0)),
            pl.BlockSpec((TN, D), lambda i: (i, 0)),
            pl.BlockSpec((TN, 1), lambda i: (i, 0)),
            pl.BlockSpec((TN, 1), lambda i: (i, 0)),
            full2(D, H), full2(D, H), full2(1, H), full2(H, O), full2(1, O),
        ],
        out_specs=pl.BlockSpec((TN, D), lambda i: (i, 0)),
        out_shape=jax.ShapeDtypeStruct((N, O), jnp.float32),
        compiler_params=pltpu.CompilerParams(
            dimension_semantics=("arbitrary",)),
        interpret=interpret,
    )(x, a0, a1, d0, d1, W1x, W1a, b1, W23, b23)


def kernel(x, edge_index, edge_attr, Wm1, bm1, Wm2, bm2, Wm3, bm3,
           Wa1, ba1, Wa2, ba2, Wa3, ba3, Wu1, bu1, Wu2, bu2, Wu3, bu3):
    N, D = x.shape
    E = edge_index.shape[1]
    DE = edge_attr.shape[1]
    C = 80
    P = 5
    EP = E // P
    EWP = EP // _NW
    NCHP = EWP // C
    NP = ((N + 255) // 256) * 256

    s5 = edge_index[0].reshape(P, _NW, NCHP, C)
    r5 = edge_index[1].reshape(P, _NW, NCHP, C)

    Wm23, bm23, wa23, Wu23, bu23 = _prep(
        Wm2, Wm3, bm2, bm3, Wa2, Wa3, Wu2, Wu3, bu2, bu3)

    gather = _make_gather(N, D, EP, C)
    ea5 = edge_attr.reshape(P, EP, DE)
    ems = []
    evs = []
    for p in range(P):
        xs, xr = gather(x, s5[p], r5[p])
        em_p, e_p = _messages(
            ea5[p], xs, xr,
            Wm1[:DE], Wm1[DE:DE + D], Wm1[DE + D:],
            bm1.reshape(1, -1), Wm23, bm23, Wa1, ba1.reshape(1, -1), wa23)
        ems.append(em_p)
        evs.append(e_p)

    e5 = jnp.stack(evs).reshape(P, _NW, NCHP, C)
    pa, pd = _make_aggregate(N, D, E, NP, C, P, P - 2, False)(
        *ems[:P - 2], e5[:P - 2], r5[:P - 2])
    pa, pd = _make_aggregate(N, D, E, NP, C, P, 1, True)(
        ems[P - 2], e5[P - 2:P - 1], r5[P - 2:P - 1], pa, pd)
    aggr_p, den_p = _make_aggregate(N, D, E, NP, C, P, 1, True)(
        ems[P - 1], e5[P - 1:], r5[P - 1:], pa, pd)

    out = _update(
        x,
        aggr_p[0, :N], aggr_p[1, :N],
        den_p[0, :N].reshape(N, 1), den_p[1, :N].reshape(N, 1),
        Wu1[:D], Wu1[D:], bu1.reshape(1, -1), Wu23, bu23.reshape(1, -1))
    return out


# speedup vs baseline: 1.0038x; 1.0038x over previous
import functools

import jax
import jax.numpy as jnp
from jax import lax
from jax.experimental import pallas as pl
from jax.experimental.pallas import tpu as pltpu
from jax.experimental.pallas import tpu_sc as plsc

_NC = 2
_NS = 16
_NW = _NC * _NS


def _prep_body(Wm2, Wm3, bm2, bm3, Wa2, Wa3, Wu2, Wu3, bu2, bu3,
               Wm23, bm23, wa23b, Wu23, bu23):
    f32 = jnp.float32
    Wm23[...] = jnp.dot(Wm2[...], Wm3[...], preferred_element_type=f32)
    bm23[...] = jnp.dot(bm2[...], Wm3[...], preferred_element_type=f32) + bm3[...]
    wa23 = jnp.dot(Wa2[...], Wa3[...], preferred_element_type=f32)
    wa23b[...] = jnp.dot(wa23, jnp.ones((1, wa23.shape[0]), f32),
                         preferred_element_type=f32)
    Wu23[...] = jnp.dot(Wu2[...], Wu3[...], preferred_element_type=f32)
    bu23[...] = jnp.dot(bu2[...], Wu3[...], preferred_element_type=f32) + bu3[...]


def _prep(Wm2, Wm3, bm2, bm3, Wa2, Wa3, Wu2, Wu3, bu2, bu3, interpret=False):
    H, M = Wm3.shape
    A = Wa2.shape[0]
    U, O = Wu3.shape
    return pl.pallas_call(
        _prep_body,
        out_shape=[
            jax.ShapeDtypeStruct((H, M), jnp.float32),
            jax.ShapeDtypeStruct((1, M), jnp.float32),
            jax.ShapeDtypeStruct((A, A), jnp.float32),
            jax.ShapeDtypeStruct((U, O), jnp.float32),
            jax.ShapeDtypeStruct((1, O), jnp.float32),
        ],
        interpret=interpret,
    )(Wm2, Wm3, bm2.reshape(1, H), bm3.reshape(1, M), Wa2, Wa3,
      Wu2, Wu3, bu2.reshape(1, U), bu3.reshape(1, O))


def _make_gather(N, D, E, C, interpret=False):
    EW = E // _NW
    NCH = EW // C
    DP = D
    dt = jnp.float32
    mesh = plsc.VectorSubcoreMesh(core_axis_name="c", subcore_axis_name="s",
                                  num_cores=_NC, num_subcores=_NS)

    @functools.partial(
        pl.kernel,
        out_type=(jax.ShapeDtypeStruct((E, DP), dt),
                  jax.ShapeDtypeStruct((E, DP), dt)),
        mesh=mesh,
        scratch_types=[
            pltpu.VMEM((NCH, C), jnp.int32),
            pltpu.VMEM((NCH, C), jnp.int32),
            pltpu.VMEM((C, DP), dt),
            pltpu.VMEM((C, DP), dt),
            pltpu.VMEM((C, DP), dt),
            pltpu.VMEM((C, DP), dt),
        ] + [pltpu.SemaphoreType.DMA] * 8,
        interpret=interpret,
    )
    def gather(x_hbm, s_hbm, r_hbm, xs_hbm, xr_hbm,
               sbuf, rbuf, a0, b0, a1, b1,
               sga0, sgb0, sga1, sgb1, swa0, swb0, swa1, swb1):
        cid = lax.axis_index("c")
        sid = lax.axis_index("s")
        wid = sid * _NC + cid
        pltpu.sync_copy(s_hbm.at[wid], sbuf)
        pltpu.sync_copy(r_hbm.at[wid], rbuf)
        base = wid * EW

        def body(p, carry):
            j0 = 2 * p
            j1 = j0 + 1

            @pl.when(p > 0)
            def _():
                pltpu.make_async_copy(a0, xs_hbm.at[pl.ds(base, C)], swa0).wait()
                pltpu.make_async_copy(b0, xr_hbm.at[pl.ds(base, C)], swb0).wait()

            ga = pltpu.async_copy(x_hbm.at[sbuf.at[j0]], a0, sga0)
            gb = pltpu.async_copy(x_hbm.at[rbuf.at[j0]], b0, sgb0)

            @pl.when(p > 0)
            def _():
                pltpu.make_async_copy(a1, xs_hbm.at[pl.ds(base, C)], swa1).wait()
                pltpu.make_async_copy(b1, xr_hbm.at[pl.ds(base, C)], swb1).wait()

            ga.wait()
            gb.wait()
            pltpu.async_copy(a0, xs_hbm.at[pl.ds(base + j0 * C, C)], swa0)
            pltpu.async_copy(b0, xr_hbm.at[pl.ds(base + j0 * C, C)], swb0)
            ga1 = pltpu.async_copy(x_hbm.at[sbuf.at[j1]], a1, sga1)
            gb1 = pltpu.async_copy(x_hbm.at[rbuf.at[j1]], b1, sgb1)
            ga1.wait()
            gb1.wait()
            pltpu.async_copy(a1, xs_hbm.at[pl.ds(base + j1 * C, C)], swa1)
            pltpu.async_copy(b1, xr_hbm.at[pl.ds(base + j1 * C, C)], swb1)
            return carry

        lax.fori_loop(0, NCH // 2, body, 0)

        pltpu.make_async_copy(a0, xs_hbm.at[pl.ds(base, C)], swa0).wait()
        pltpu.make_async_copy(b0, xr_hbm.at[pl.ds(base, C)], swb0).wait()
        pltpu.make_async_copy(a1, xs_hbm.at[pl.ds(base, C)], swa1).wait()
        pltpu.make_async_copy(b1, xr_hbm.at[pl.ds(base, C)], swb1).wait()

        if NCH % 2:
            jt = NCH - 1
            ga = pltpu.async_copy(x_hbm.at[sbuf.at[jt]], a0, sga0)
            gb = pltpu.async_copy(x_hbm.at[rbuf.at[jt]], b0, sgb0)
            ga.wait()
            gb.wait()
            pltpu.sync_copy(a0, xs_hbm.at[pl.ds(base + jt * C, C)])
            pltpu.sync_copy(b0, xr_hbm.at[pl.ds(base + jt * C, C)])

    return gather


def _msg_body(ea, xs, xr, We, Ws, Wr, bm1, Wm23, bm23, Wa1, ba1, wa23b,
              em_ref, e_ref):
    f32 = jnp.float32
    dflt = jax.lax.Precision.DEFAULT
    dot = lambda a, b: jnp.dot(a, b, preferred_element_type=f32,
                               precision=dflt)
    h = dot(ea[...], We[...])
    h = h + dot(xs[...], Ws[...])
    h = h + dot(xr[...], Wr[...])
    h = jnp.maximum(h + bm1[...], 0.0)
    msgs = dot(h, Wm23[...]) + bm23[...]
    g1 = jnp.maximum(dot(msgs, Wa1[...]) + ba1[...], 0.0)
    gate = dot(g1, wa23b[...])
    ev = jnp.exp(gate)
    em_ref[...] = msgs * ev
    e_ref[...] = ev[:, 0]


def _messages(ea, xs, xr, We, Ws, Wr, bm1, Wm23, bm23, Wa1, ba1, wa23,
              interpret=False):
    E, DE = ea.shape
    DP = xs.shape[1]
    D = Wa1.shape[0]
    H = Ws.shape[1]
    TE = 512
    grid = E // TE
    full2 = lambda a, b: pl.BlockSpec((a, b), lambda i: (0, 0))
    return pl.pallas_call(
        _msg_body,
        grid=(grid,),
        in_specs=[
            pl.BlockSpec((TE, DE), lambda i: (i, 0)),
            pl.BlockSpec((TE, DP), lambda i: (i, 0)),
            pl.BlockSpec((TE, DP), lambda i: (i, 0)),
            full2(DE, H), full2(D, H), full2(D, H), full2(1, H),
            full2(H, D), full2(1, D), full2(D, D), full2(1, D), full2(D, D),
        ],
        out_specs=[
            pl.BlockSpec((TE, D), lambda i: (i, 0)),
            pl.BlockSpec((TE,), lambda i: (i,)),
        ],
        out_shape=[
            jax.ShapeDtypeStruct((E, D), jnp.float32),
            jax.ShapeDtypeStruct((E,), jnp.float32),
        ],
        compiler_params=pltpu.CompilerParams(
            dimension_semantics=("arbitrary",)),
        interpret=interpret,
    )(ea, xs, xr, We, Ws, Wr, bm1, Wm23, bm23, Wa1, ba1, wa23)


def _make_aggregate(N, D, E, NP, C, P, nph, init, interpret=False):
    EP = E // P
    EWP = EP // _NW
    NB = NP // _NS
    ZR = 64
    B2 = EWP // C
    mesh = plsc.VectorSubcoreMesh(core_axis_name="c", subcore_axis_name="s",
                                  num_cores=_NC, num_subcores=_NS)

    @functools.partial(
        pl.kernel,
        out_type=(jax.ShapeDtypeStruct((_NC, NP, D), jnp.float32),
                  jax.ShapeDtypeStruct((_NC, NP), jnp.float32)),
        mesh=mesh,
        scratch_types=[
            pltpu.VMEM((B2, C), jnp.int32),
            pltpu.VMEM((B2, C), jnp.float32),
            pltpu.VMEM((C, D), jnp.float32),
            pltpu.VMEM((C, D), jnp.float32),
            pltpu.VMEM((ZR, D), jnp.float32),
            pltpu.VMEM((NB,), jnp.float32),
            pltpu.VMEM_SHARED((NP, D), jnp.float32),
            pltpu.VMEM_SHARED((NP,), jnp.float32),
            pltpu.SemaphoreType.DMA,
            pltpu.SemaphoreType.DMA,
            pltpu.SemaphoreType.DMA,
        ],
        interpret=interpret,
    )
    def agg(*refs):
        ems = refs[:nph]
        e_hbm = refs[nph]
        r_hbm = refs[nph + 1]
        k0 = nph + 2
        if init:
            initA = refs[k0]
            initD = refs[k0 + 1]
            k0 += 2
        (aggr_out, den_out, rbuf, ebuf, r0, r1, zrows, dstage,
         aggr_sh, den_sh, sl0, sl1, se) = refs[k0:]
        cid = lax.axis_index("c")
        sid = lax.axis_index("s")
        wid = sid * _NC + cid
        zero16 = jnp.zeros((16,), jnp.float32)

        tb = sid * NB
        if init:
            pltpu.sync_copy(initD.at[cid, pl.ds(tb, NB)], dstage)
            pltpu.sync_copy(dstage, den_sh.at[pl.ds(tb, NB)])
            for k in range(NB // ZR):
                pltpu.sync_copy(initA.at[cid, pl.ds(tb + k * ZR, ZR)], zrows)
                pltpu.sync_copy(zrows, aggr_sh.at[pl.ds(tb + k * ZR, ZR)])
        else:
            def z1(i, c):
                dstage[pl.ds(i * 16, 16)] = zero16
                return c
            lax.fori_loop(0, NB // 16, z1, 0)

            def z2(t, c):
                zrows[t // (D // 16), pl.ds((t % (D // 16)) * 16, 16)] = zero16
                return c
            lax.fori_loop(0, ZR * (D // 16), z2, 0)

            for k in range(NB // ZR):
                pltpu.sync_copy(zrows, aggr_sh.at[pl.ds(tb + k * ZR, ZR)])
            pltpu.sync_copy(dstage, den_sh.at[pl.ds(tb, NB)])
        plsc.subcore_barrier()

        base = wid * EWP

        for b, em_hbm in enumerate(ems):
            pltpu.sync_copy(r_hbm.at[b, wid], rbuf)
            pltpu.sync_copy(e_hbm.at[b, wid], ebuf)

            def pair(p, c2, em_hbm=em_hbm):
                j0 = 2 * p
                j1 = j0 + 1
                ld0 = pltpu.async_copy(
                    em_hbm.at[pl.ds(base + j0 * C, C)], r0, sl0)
                ld1 = pltpu.async_copy(
                    em_hbm.at[pl.ds(base + j1 * C, C)], r1, sl1)
                pltpu.async_copy(ebuf.at[j0], den_sh.at[rbuf.at[j0]], se,
                                 add=True)
                pltpu.async_copy(ebuf.at[j1], den_sh.at[rbuf.at[j1]], se,
                                 add=True)
                ld0.wait()
                pltpu.sync_copy(r0, aggr_sh.at[rbuf.at[j0]], add=True)
                ld1.wait()
                pltpu.sync_copy(r1, aggr_sh.at[rbuf.at[j1]], add=True)
                return c2

            lax.fori_loop(0, B2 // 2, pair, 0)
            if B2 % 2:
                jt = B2 - 1
                pltpu.async_copy(ebuf.at[jt], den_sh.at[rbuf.at[jt]], se,
                                 add=True)
                pltpu.sync_copy(em_hbm.at[pl.ds(base + jt * C, C)], r0)
                pltpu.sync_copy(r0, aggr_sh.at[rbuf.at[jt]], add=True)

            def drain(k, c2):
                pltpu.make_async_copy(
                    ebuf.at[0], den_sh.at[rbuf.at[0]], se).wait()
                return c2

            lax.fori_loop(0, B2, drain, 0)
        plsc.subcore_barrier()

        pltpu.sync_copy(den_sh.at[pl.ds(tb, NB)], dstage)
        pltpu.sync_copy(dstage, den_out.at[cid, pl.ds(tb, NB)])
        for k in range(NB // ZR):
            pltpu.sync_copy(aggr_sh.at[pl.ds(tb + k * ZR, ZR)], zrows)
            pltpu.sync_copy(zrows, aggr_out.at[cid, pl.ds(tb + k * ZR, ZR)])

    return agg


def _upd_body(x, a0, a1, d0, d1, W1x, W1a, b1, W23, b23, o_ref):
    f32 = jnp.float32
    d = d0[...] + d1[...]
    pos = d > 0.0
    inv = jnp.where(pos, 1.0 / jnp.where(pos, d, 1.0), 0.0)
    aggr = (a0[...] + a1[...]) * inv
    u = jnp.dot(x[...], W1x[...], preferred_element_type=f32)
    u = u + jnp.dot(aggr, W1a[...], preferred_element_type=f32)
    u = jnp.maximum(u + b1[...], 0.0)
    o_ref[...] = jnp.dot(u, W23[...], preferred_element_type=f32) + b23[...]


def _update(x, a0, a1, d0, d1, W1x, W1a, b1, W23, b23, interpret=False):
    N, D = x.shape
    H = W1x.shape[1]
    O = W23.shape[1]
    TN = 1000
    grid = N // TN
    full2 = lambda a, b: pl.BlockSpec((a, b), lambda i: (0, 0))
    return pl.pallas_call(
        _upd_body,
        grid=(grid,),
        in_specs=[
            pl.BlockSpec((TN, D), lambda i: (i, 0)),
            pl.BlockSpec((TN, D), lambda i: (i, 0)),
            pl.BlockSpec((TN, D), lambda i: (i, 0)),
            pl.BlockSpec((TN, 1), lambda i: (i, 0)),
            pl.BlockSpec((TN, 1), lambda i: (i, 0)),
            full2(D, H), full2(D, H), full2(1, H), full2(H, O), full2(1, O),
        ],
        out_specs=pl.BlockSpec((TN, D), lambda i: (i, 0)),
        out_shape=jax.ShapeDtypeStruct((N, O), jnp.float32),
        compiler_params=pltpu.CompilerParams(
            dimension_semantics=("arbitrary",)),
        interpret=interpret,
    )(x, a0, a1, d0, d1, W1x, W1a, b1, W23, b23)


def kernel(x, edge_index, edge_attr, Wm1, bm1, Wm2, bm2, Wm3, bm3,
           Wa1, ba1, Wa2, ba2, Wa3, ba3, Wu1, bu1, Wu2, bu2, Wu3, bu3):
    N, D = x.shape
    E = edge_index.shape[1]
    DE = edge_attr.shape[1]
    C = 80
    P = 5
    EP = E // P
    EWP = EP // _NW
    NCHP = EWP // C
    NP = ((N + 255) // 256) * 256

    s5 = edge_index[0].reshape(P, _NW, NCHP, C)
    r5 = edge_index[1].reshape(P, _NW, NCHP, C)

    Wm23, bm23, wa23, Wu23, bu23 = _prep(
        Wm2, Wm3, bm2, bm3, Wa2, Wa3, Wu2, Wu3, bu2, bu3)

    gather = _make_gather(N, D, EP, C)
    ea5 = edge_attr.reshape(P, EP, DE)
    ems = []
    evs = []
    for p in range(P):
        xs, xr = gather(x, s5[p], r5[p])
        em_p, e_p = _messages(
            ea5[p], xs, xr,
            Wm1[:DE], Wm1[DE:DE + D], Wm1[DE + D:],
            bm1.reshape(1, -1), Wm23, bm23, Wa1, ba1.reshape(1, -1), wa23)
        ems.append(em_p)
        evs.append(e_p)

    e5 = jnp.stack(evs).reshape(P, _NW, NCHP, C)
    pa, pd = _make_aggregate(N, D, E, NP, C, P, P - 1, False)(
        *ems[:P - 1], e5[:P - 1], r5[:P - 1])
    aggr_p, den_p = _make_aggregate(N, D, E, NP, C, P, 1, True)(
        ems[P - 1], e5[P - 1:], r5[P - 1:], pa, pd)

    out = _update(
        x,
        aggr_p[0, :N], aggr_p[1, :N],
        den_p[0, :N].reshape(N, 1), den_p[1, :N].reshape(N, 1),
        Wu1[:D], Wu1[D:], bu1.reshape(1, -1), Wu23, bu23.reshape(1, -1))
    return out
